# baseline (speedup 1.0000x reference)
import functools

import jax
import jax.numpy as jnp
from jax import lax
from jax.experimental import pallas as pl
from jax.experimental.pallas import tpu as pltpu
from jax.experimental.pallas import tpu_sc as plsc

_CHUNK = 160
_NBUF = 5


@functools.cache
def _make_gather(B, V, D):
    info = plsc.get_sparse_core_info()
    nw = info.num_cores * info.num_subcores
    b_per_w = B // nw
    n_chunks = b_per_w // _CHUNK
    n_rings = n_chunks // _NBUF
    mesh = plsc.VectorSubcoreMesh(core_axis_name="c", subcore_axis_name="s")

    @functools.partial(
        pl.kernel,
        out_type=jax.ShapeDtypeStruct((B, D), jnp.float32),
        mesh=mesh,
        compiler_params=pltpu.CompilerParams(use_tc_tiling_on_sc=False),
        scratch_types=[
            pltpu.VMEM((b_per_w,), jnp.int32),
            pltpu.VMEM((_NBUF, _CHUNK, D), jnp.float32),
            pltpu.SemaphoreType.DMA((_NBUF,)),
            pltpu.SemaphoreType.DMA((_NBUF,)),
        ],
    )
    def gather_kernel(idx_hbm, table_hbm, out_hbm, idx_v, rows_v, gsem, wsem):
        wid = lax.axis_index("s") * info.num_cores + lax.axis_index("c")
        base = wid * b_per_w
        pltpu.sync_copy(idx_hbm.at[pl.ds(base, b_per_w)], idx_v)

        def fire(j, b):
            pltpu.async_copy(
                table_hbm.at[idx_v.at[pl.ds(j * _CHUNK, _CHUNK)]],
                rows_v.at[b],
                gsem.at[b],
            )

        def drain_fire_wb(j, b):
            pltpu.make_async_copy(
                table_hbm.at[idx_v.at[pl.ds(0, _CHUNK)]], rows_v.at[b], gsem.at[b]
            ).wait()
            pltpu.async_copy(
                rows_v.at[b], out_hbm.at[pl.ds(base + j * _CHUNK, _CHUNK)], wsem.at[b]
            )

        def wait_wb(j, b):
            pltpu.make_async_copy(
                rows_v.at[b], out_hbm.at[pl.ds(base + j * _CHUNK, _CHUNK)], wsem.at[b]
            ).wait()

        for b in range(_NBUF):
            fire(b, b)

        def ring_body(g, carry):
            jbase = g * _NBUF
            for b in range(_NBUF):
                drain_fire_wb(jbase + b, b)
            for b in range(_NBUF):
                wait_wb(jbase + b, b)
                fire(jbase + _NBUF + b, b)
            return carry

        lax.fori_loop(0, n_rings - 1, ring_body, 0)

        jbase = (n_rings - 1) * _NBUF
        for b in range(_NBUF):
            drain_fire_wb(jbase + b, b)
        for b in range(_NBUF):
            wait_wb(jbase + b, b)

    return gather_kernel


@jax.jit
def kernel(inputs, table):
    batch, seq = inputs.shape
    vocab, embed = table.shape
    idx = inputs.T.reshape(-1)
    out = _make_gather(idx.shape[0], vocab, embed)(idx, table)
    return out.reshape(seq, batch, embed).transpose(1, 0, 2)


# speedup vs baseline: 1.0295x; 1.0295x over previous
import functools

import jax
import jax.numpy as jnp
from jax import lax
from jax.experimental import pallas as pl
from jax.experimental.pallas import tpu as pltpu
from jax.experimental.pallas import tpu_sc as plsc

_CHUNK = 128
_NBUF = 4


@functools.cache
def _make_gather(batch, seq, V, D):
    B = batch * seq
    info = plsc.get_sparse_core_info()
    nw = info.num_cores * info.num_subcores
    cols_per_w = batch // nw
    b_per_w = cols_per_w * seq
    cpr = cols_per_w // _CHUNK
    n_chunks = b_per_w // _CHUNK
    n_rings = n_chunks // _NBUF
    mesh = plsc.VectorSubcoreMesh(core_axis_name="c", subcore_axis_name="s")

    @functools.partial(
        pl.kernel,
        out_type=jax.ShapeDtypeStruct((B, D), jnp.float32),
        mesh=mesh,
        compiler_params=pltpu.CompilerParams(use_tc_tiling_on_sc=False),
        scratch_types=[
            pltpu.VMEM((seq, cols_per_w), jnp.int32),
            pltpu.VMEM((_NBUF, _CHUNK, D), jnp.float32),
            pltpu.SemaphoreType.DMA((_NBUF,)),
            pltpu.SemaphoreType.DMA((_NBUF,)),
        ],
    )
    def gather_kernel(idx_hbm, table_hbm, out_hbm, idx_v, rows_v, gsem, wsem):
        wid = lax.axis_index("s") * info.num_cores + lax.axis_index("c")
        cbase = wid * cols_per_w
        pltpu.sync_copy(idx_hbm.at[:, pl.ds(cbase, cols_per_w)], idx_v)

        def out_slice(j):
            s, c = j // cpr, j % cpr
            return out_hbm.at[pl.ds(s * batch + cbase + c * _CHUNK, _CHUNK)]

        def idx_slice(j):
            s, c = j // cpr, j % cpr
            return idx_v.at[s, pl.ds(c * _CHUNK, _CHUNK)]

        def fire(j, b):
            pltpu.async_copy(
                table_hbm.at[idx_slice(j)], rows_v.at[b], gsem.at[b]
            )

        def drain_fire_wb(j, b):
            pltpu.make_async_copy(
                table_hbm.at[idx_slice(0)], rows_v.at[b], gsem.at[b]
            ).wait()
            pltpu.async_copy(rows_v.at[b], out_slice(j), wsem.at[b])

        def wait_wb(j, b):
            pltpu.make_async_copy(rows_v.at[b], out_slice(j), wsem.at[b]).wait()

        for b in range(_NBUF):
            fire(b, b)

        def ring_body(g, carry):
            jbase = g * _NBUF
            for b in range(_NBUF):
                drain_fire_wb(jbase + b, b)
            for b in range(_NBUF):
                wait_wb(jbase + b, b)
                fire(jbase + _NBUF + b, b)
            return carry

        lax.fori_loop(0, n_rings - 1, ring_body, 0)

        jbase = (n_rings - 1) * _NBUF
        for b in range(_NBUF):
            drain_fire_wb(jbase + b, b)
        for b in range(_NBUF):
            wait_wb(jbase + b, b)

    return gather_kernel


@jax.jit
def kernel(inputs, table):
    batch, seq = inputs.shape
    vocab, embed = table.shape
    out = _make_gather(batch, seq, vocab, embed)(inputs.T, table)
    return out.reshape(seq, batch, embed).transpose(1, 0, 2)
